# baseline (speedup 1.0000x reference)
import functools

import jax
import jax.numpy as jnp
from jax import lax
from jax.experimental import pallas as pl
from jax.experimental.pallas import tpu as pltpu
from jax.experimental.pallas import tpu_sc as plsc

N_NODES = 10000
N_EDGES = 160000
POS_DIM = 3
FEATS = 256

NW = 32
CHUNK = 128
NCHUNK = 40
E_W = CHUNK * NCHUNK
E_PAD = NW * E_W


def _ab_body(x_ref, w6_ref, out_ref):
    x = x_ref[...]
    ab = jnp.dot(x, w6_ref[...], preferred_element_type=jnp.float32)
    pad = jnp.zeros((x.shape[0], 7), jnp.float32)
    out_ref[...] = jnp.concatenate([x[:, 0:POS_DIM], ab, pad], axis=1)


def _node_proj(x, w6p):
    blk = 2000
    grid = (N_NODES + blk - 1) // blk
    return pl.pallas_call(
        _ab_body,
        grid=(grid,),
        in_specs=[pl.BlockSpec((blk, POS_DIM + FEATS), lambda i: (i, 0)),
                  pl.BlockSpec((POS_DIM + FEATS, 6), lambda i: (0, 0))],
        out_specs=pl.BlockSpec((blk, 16), lambda i: (i, 0)),
        out_shape=jax.ShapeDtypeStruct((N_NODES, 16), jnp.float32),
    )(x, w6p)


BLKE = 16000
GRID_E = N_EDGES // BLKE


def _ct_body(attr_hbm, wa_ref, b_ref, out_hbm, abuf, cbuf, sem_i, sem_o):
    i = pl.program_id(0)
    cp = pltpu.make_async_copy(
        attr_hbm.at[pl.ds(i * BLKE, BLKE), :], abuf, sem_i)
    cp.start()
    cp.wait()
    cbuf[...] = jnp.dot(abuf[...], wa_ref[...],
                        preferred_element_type=jnp.float32) + b_ref[...]
    cp2 = pltpu.make_async_copy(
        cbuf, out_hbm.at[pl.ds(i * BLKE, BLKE), :], sem_o)
    cp2.start()
    cp2.wait()


def _edge_proj_t(attr, wa, eb1):
    return pl.pallas_call(
        _ct_body,
        grid=(GRID_E,),
        in_specs=[pl.BlockSpec(memory_space=pl.ANY),
                  pl.BlockSpec((16, 3), lambda i: (0, 0)),
                  pl.BlockSpec((1, 3), lambda i: (0, 0))],
        out_specs=pl.BlockSpec(memory_space=pl.ANY),
        out_shape=jax.ShapeDtypeStruct((E_PAD, 3), jnp.float32),
        scratch_shapes=[pltpu.VMEM((BLKE, 16), jnp.float32),
                        pltpu.VMEM((BLKE, 3), jnp.float32),
                        pltpu.SemaphoreType.DMA,
                        pltpu.SemaphoreType.DMA],
    )(attr, wa, eb1.reshape(1, 3))


def _sc_body(row_hbm, col_hbm, table_hbm, ct_hbm, prm_hbm, out_hbm,
             rowi, coli, cst, prm, rb0, cb0, rb1, cb1,
             a0, a1, a2, a3, a4, a5, sem_r0, sem_c0, sem_r1, sem_c1):
    nc = 2
    wid = lax.axis_index("s") * nc + lax.axis_index("c")

    pltpu.sync_copy(row_hbm.at[wid], rowi)
    pltpu.sync_copy(col_hbm.at[wid], coli)
    pltpu.async_copy(table_hbm.at[rowi.at[0]], rb0, sem_r0)
    pltpu.async_copy(table_hbm.at[coli.at[0]], cb0, sem_c0)
    pltpu.sync_copy(ct_hbm.at[pl.ds(wid * E_W, E_W), :], cst)
    pltpu.sync_copy(prm_hbm, prm)

    zeros16 = jnp.zeros((16,), jnp.float32)

    @pl.loop(0, N_NODES // 16)
    def _zero(i):
        sl = pl.ds(i * 16, 16)
        a0[sl] = zeros16
        a1[sl] = zeros16
        a2[sl] = zeros16
        a3[sl] = zeros16
        a4[sl] = zeros16
        a5[sl] = zeros16

    li0 = lax.iota(jnp.int32, 16)
    cols = [jnp.full((16,), k, jnp.int32) for k in range(16)]
    e_base = wid * E_W

    def _compute(j, rbuf, cbuf):
        for i in range(8):
            li = li0 + (i * 16)
            ridx = rowi[j, pl.ds(i * 16, 16)]
            pxr = plsc.load_gather(rbuf, [li, cols[0]])
            pyr = plsc.load_gather(rbuf, [li, cols[1]])
            pzr = plsc.load_gather(rbuf, [li, cols[2]])
            av0 = plsc.load_gather(rbuf, [li, cols[3]])
            av1 = plsc.load_gather(rbuf, [li, cols[4]])
            av2 = plsc.load_gather(rbuf, [li, cols[5]])
            pxc = plsc.load_gather(cbuf, [li, cols[0]])
            pyc = plsc.load_gather(cbuf, [li, cols[1]])
            pzc = plsc.load_gather(cbuf, [li, cols[2]])
            bv0 = plsc.load_gather(cbuf, [li, cols[6]])
            bv1 = plsc.load_gather(cbuf, [li, cols[7]])
            bv2 = plsc.load_gather(cbuf, [li, cols[8]])
            eli = li0 + (j * CHUNK + i * 16)
            c0 = plsc.load_gather(cst, [eli, cols[0]])
            c1 = plsc.load_gather(cst, [eli, cols[1]])
            c2 = plsc.load_gather(cst, [eli, cols[2]])

            rx = pxr - pxc
            ry = pyr - pyc
            rz = pzr - pzc
            dist = rx * rx + ry * ry + rz * rz

            pre0 = av0 + bv0 + c0 + dist * prm[12]
            pre1 = av1 + bv1 + c1 + dist * prm[13]
            pre2 = av2 + bv2 + c2 + dist * prm[14]
            m0 = pre0 / (1.0 + jnp.exp(-pre0))
            m1 = pre1 / (1.0 + jnp.exp(-pre1))
            m2 = pre2 / (1.0 + jnp.exp(-pre2))

            q0 = m0 * prm[0] + m1 * prm[3] + m2 * prm[6] + prm[9]
            q1 = m0 * prm[1] + m1 * prm[4] + m2 * prm[7] + prm[10]
            q2 = m0 * prm[2] + m1 * prm[5] + m2 * prm[8] + prm[11]
            n0 = q0 / (1.0 + jnp.exp(-q0))
            n1 = q1 / (1.0 + jnp.exp(-q1))
            n2 = q2 / (1.0 + jnp.exp(-q2))

            mask = (li + (e_base + j * CHUNK)) < N_EDGES
            plsc.addupdate_scatter(a0, [ridx], n0, mask=mask)
            plsc.addupdate_scatter(a1, [ridx], n1, mask=mask)
            plsc.addupdate_scatter(a2, [ridx], n2, mask=mask)
            plsc.addupdate_scatter(a3, [ridx], n0 * rx, mask=mask)
            plsc.addupdate_scatter(a4, [ridx], n1 * ry, mask=mask)
            plsc.addupdate_scatter(a5, [ridx], n2 * rz, mask=mask)

    @pl.loop(0, NCHUNK // 2)
    def _pair(i):
        j0 = i * 2
        j1 = j0 + 1
        pltpu.async_copy(table_hbm.at[rowi.at[j1]], rb1, sem_r1)
        pltpu.async_copy(table_hbm.at[coli.at[j1]], cb1, sem_c1)
        pltpu.make_async_copy(table_hbm.at[rowi.at[j0]], rb0, sem_r0).wait()
        pltpu.make_async_copy(table_hbm.at[coli.at[j0]], cb0, sem_c0).wait()
        _compute(j0, rb0, cb0)
        jn = jnp.minimum(j0 + 2, NCHUNK - 1)
        pltpu.async_copy(table_hbm.at[rowi.at[jn]], rb0, sem_r0)
        pltpu.async_copy(table_hbm.at[coli.at[jn]], cb0, sem_c0)
        pltpu.make_async_copy(table_hbm.at[rowi.at[j1]], rb1, sem_r1).wait()
        pltpu.make_async_copy(table_hbm.at[coli.at[j1]], cb1, sem_c1).wait()
        _compute(j1, rb1, cb1)

    pltpu.make_async_copy(table_hbm.at[rowi.at[0]], rb0, sem_r0).wait()
    pltpu.make_async_copy(table_hbm.at[coli.at[0]], cb0, sem_c0).wait()

    pltpu.sync_copy(a0, out_hbm.at[wid, 0])
    pltpu.sync_copy(a1, out_hbm.at[wid, 1])
    pltpu.sync_copy(a2, out_hbm.at[wid, 2])
    pltpu.sync_copy(a3, out_hbm.at[wid, 3])
    pltpu.sync_copy(a4, out_hbm.at[wid, 4])
    pltpu.sync_copy(a5, out_hbm.at[wid, 5])


def _sc_edges(rowp, colp, table, ct, prm):
    mesh = plsc.VectorSubcoreMesh(core_axis_name="c", subcore_axis_name="s",
                                  num_cores=2, num_subcores=16)
    f = pl.kernel(
        _sc_body,
        out_type=jax.ShapeDtypeStruct((NW, 6, N_NODES), jnp.float32),
        mesh=mesh,
        compiler_params=pltpu.CompilerParams(needs_layout_passes=False,
                                             use_tc_tiling_on_sc=False),
        scratch_types=[
            pltpu.VMEM((NCHUNK, CHUNK), jnp.int32),
            pltpu.VMEM((NCHUNK, CHUNK), jnp.int32),
            pltpu.VMEM((E_W, 3), jnp.float32),
            pltpu.VMEM((16, 16), jnp.float32),
            pltpu.VMEM((CHUNK, 16), jnp.float32),
            pltpu.VMEM((CHUNK, 16), jnp.float32),
            pltpu.VMEM((CHUNK, 16), jnp.float32),
            pltpu.VMEM((CHUNK, 16), jnp.float32),
            pltpu.VMEM((N_NODES,), jnp.float32),
            pltpu.VMEM((N_NODES,), jnp.float32),
            pltpu.VMEM((N_NODES,), jnp.float32),
            pltpu.VMEM((N_NODES,), jnp.float32),
            pltpu.VMEM((N_NODES,), jnp.float32),
            pltpu.VMEM((N_NODES,), jnp.float32),
            pltpu.SemaphoreType.DMA,
            pltpu.SemaphoreType.DMA,
            pltpu.SemaphoreType.DMA,
            pltpu.SemaphoreType.DMA,
        ],
    )
    return f(rowp, colp, table, ct, prm)


def _node_body(x_ref, parts_ref, w1x_ref, w1m_ref, b1_ref,
               w2_ref, b2_ref, out_ref):
    acc = jnp.sum(parts_ref[...], axis=0)
    aggm_t = acc[0:3]
    aggp_t = acc[3:6]
    x = x_ref[...]
    z = (jnp.dot(x, w1x_ref[...], preferred_element_type=jnp.float32)
         + lax.dot_general(aggm_t, w1m_ref[...], (((0,), (0,)), ((), ())),
                           preferred_element_type=jnp.float32)
         + b1_ref[...])
    h2 = z * jax.nn.sigmoid(z)
    d2 = jnp.dot(h2, w2_ref[...],
                 preferred_element_type=jnp.float32) + b2_ref[...]
    dpos = lax.dot_general(aggp_t * 0.1, jnp.eye(3, dtype=jnp.float32),
                           (((0,), (0,)), ((), ())),
                           preferred_element_type=jnp.float32)
    out_ref[...] = x + jnp.concatenate([dpos, d2], axis=1)


def _node_mlp(x, parts, w1x, w1m, b1, w2, b2):
    blk = 2048
    grid = (N_NODES + blk - 1) // blk
    d_all = POS_DIM + FEATS
    return pl.pallas_call(
        _node_body,
        grid=(grid,),
        in_specs=[
            pl.BlockSpec((blk, d_all), lambda i: (i, 0)),
            pl.BlockSpec((NW, 6, blk), lambda i: (0, 0, i)),
            pl.BlockSpec((d_all, FEATS), lambda i: (0, 0)),
            pl.BlockSpec((3, FEATS), lambda i: (0, 0)),
            pl.BlockSpec((1, FEATS), lambda i: (0, 0)),
            pl.BlockSpec((FEATS, FEATS), lambda i: (0, 0)),
            pl.BlockSpec((1, FEATS), lambda i: (0, 0)),
        ],
        out_specs=pl.BlockSpec((blk, d_all), lambda i: (i, 0)),
        out_shape=jax.ShapeDtypeStruct((N_NODES, d_all), jnp.float32),
    )(x, parts, w1x, w1m, b1, w2, b2)


@jax.jit
def kernel(x, edge_index, edge_attr, batch, eW1, eb1, eW2, eb2,
           nW1, nb1, nW2, nb2):
    row = edge_index[0].astype(jnp.int32)
    col = edge_index[1].astype(jnp.int32)

    w6 = jnp.concatenate([eW1[:FEATS], eW1[FEATS:2 * FEATS]], axis=1)
    w6p = jnp.concatenate([jnp.zeros((POS_DIM, 6), jnp.float32), w6], axis=0)
    table = _node_proj(x, w6p)

    ct = _edge_proj_t(edge_attr, eW1[2 * FEATS:2 * FEATS + 16], eb1)

    pad = E_PAD - N_EDGES
    rowp = jnp.pad(row, (0, pad)).reshape(NW, NCHUNK, CHUNK)
    colp = jnp.pad(col, (0, pad)).reshape(NW, NCHUNK, CHUNK)

    p = jnp.concatenate([eW2.reshape(-1), eb2, eW1[2 * FEATS + 16]])
    prm = jnp.broadcast_to(jnp.pad(p, (0, 1))[:, None], (16, 16))

    parts = _sc_edges(rowp, colp, table, ct, prm)

    w1x = jnp.concatenate([jnp.zeros((POS_DIM, FEATS), jnp.float32),
                           nW1[:FEATS]], axis=0)
    return _node_mlp(x, parts, w1x, nW1[FEATS:], nb1.reshape(1, -1),
                     nW2, nb2.reshape(1, -1))


# speedup vs baseline: 1.6421x; 1.6421x over previous
import functools

import jax
import jax.numpy as jnp
from jax import lax
from jax.experimental import pallas as pl
from jax.experimental.pallas import tpu as pltpu
from jax.experimental.pallas import tpu_sc as plsc

N_NODES = 10000
N_EDGES = 160000
POS_DIM = 3
FEATS = 256

NW = 32
CHUNK = 128
NCHUNK = 40
E_W = CHUNK * NCHUNK
E_PAD = NW * E_W


def _ab_body(x_ref, w6_ref, out_ref):
    x = x_ref[...]
    ab = jnp.dot(x, w6_ref[...], preferred_element_type=jnp.float32)
    pad = jnp.zeros((x.shape[0], 7), jnp.float32)
    out_ref[...] = jnp.concatenate([x[:, 0:POS_DIM], ab, pad], axis=1)


def _node_proj(x, w6p):
    blk = 2000
    grid = (N_NODES + blk - 1) // blk
    return pl.pallas_call(
        _ab_body,
        grid=(grid,),
        in_specs=[pl.BlockSpec((blk, POS_DIM + FEATS), lambda i: (i, 0)),
                  pl.BlockSpec((POS_DIM + FEATS, 6), lambda i: (0, 0))],
        out_specs=pl.BlockSpec((blk, 16), lambda i: (i, 0)),
        out_shape=jax.ShapeDtypeStruct((N_NODES, 16), jnp.float32),
    )(x, w6p)


A_ROWS = N_EDGES * 16 // 128
A_COLS = E_PAD * 16 // 128


def _ct_body(wp_ref, attr_ref, b_ref, out_ref):
    out_ref[...] = lax.dot_general(
        wp_ref[...], attr_ref[...], (((0,), (1,)), ((), ())),
        preferred_element_type=jnp.float32) + b_ref[...]


def _edge_proj_t(attr2d, wp, b24):
    blk = 2048
    grid = (A_ROWS + blk - 1) // blk
    return pl.pallas_call(
        _ct_body,
        grid=(grid,),
        in_specs=[pl.BlockSpec((128, 24), lambda i: (0, 0)),
                  pl.BlockSpec((blk, 128), lambda i: (i, 0)),
                  pl.BlockSpec((24, 1), lambda i: (0, 0))],
        out_specs=pl.BlockSpec((24, blk), lambda i: (0, i)),
        out_shape=jax.ShapeDtypeStruct((24, A_COLS), jnp.float32),
    )(wp, attr2d, b24)


CW = E_W // 8


def _sc_body(row_hbm, col_hbm, table_hbm, ct_hbm, prm_hbm, out_hbm,
             rowi, coli, cst, prm, rb0, cb0, rb1, cb1,
             a0, a1, a2, a3, a4, a5, sem_r0, sem_c0, sem_r1, sem_c1):
    nc = 2
    wid = lax.axis_index("s") * nc + lax.axis_index("c")

    pltpu.sync_copy(row_hbm.at[wid], rowi)
    pltpu.sync_copy(col_hbm.at[wid], coli)
    pltpu.async_copy(table_hbm.at[rowi.at[0]], rb0, sem_r0)
    pltpu.async_copy(table_hbm.at[coli.at[0]], cb0, sem_c0)
    pltpu.sync_copy(ct_hbm.at[:, pl.ds(wid * CW, CW)], cst)
    pltpu.sync_copy(prm_hbm, prm)

    zeros16 = jnp.zeros((16,), jnp.float32)

    @pl.loop(0, N_NODES // 16)
    def _zero(i):
        sl = pl.ds(i * 16, 16)
        a0[sl] = zeros16
        a1[sl] = zeros16
        a2[sl] = zeros16
        a3[sl] = zeros16
        a4[sl] = zeros16
        a5[sl] = zeros16

    li0 = lax.iota(jnp.int32, 16)
    cols = [jnp.full((16,), k, jnp.int32) for k in range(16)]
    e_base = wid * E_W
    row3 = (li0 & 7) * 3
    cshift = li0 >> 3

    def _compute(j, rbuf, cbuf):
        for i in range(8):
            li = li0 + (i * 16)
            ridx = rowi[j, pl.ds(i * 16, 16)]
            pxr = plsc.load_gather(rbuf, [li, cols[0]])
            pyr = plsc.load_gather(rbuf, [li, cols[1]])
            pzr = plsc.load_gather(rbuf, [li, cols[2]])
            av0 = plsc.load_gather(rbuf, [li, cols[3]])
            av1 = plsc.load_gather(rbuf, [li, cols[4]])
            av2 = plsc.load_gather(rbuf, [li, cols[5]])
            pxc = plsc.load_gather(cbuf, [li, cols[0]])
            pyc = plsc.load_gather(cbuf, [li, cols[1]])
            pzc = plsc.load_gather(cbuf, [li, cols[2]])
            bv0 = plsc.load_gather(cbuf, [li, cols[6]])
            bv1 = plsc.load_gather(cbuf, [li, cols[7]])
            bv2 = plsc.load_gather(cbuf, [li, cols[8]])
            colv = cshift + (j * (CHUNK // 8) + i * 2)
            c0 = plsc.load_gather(cst, [row3, colv])
            c1 = plsc.load_gather(cst, [row3 + 1, colv])
            c2 = plsc.load_gather(cst, [row3 + 2, colv])

            rx = pxr - pxc
            ry = pyr - pyc
            rz = pzr - pzc
            dist = rx * rx + ry * ry + rz * rz

            pre0 = av0 + bv0 + c0 + dist * prm[12]
            pre1 = av1 + bv1 + c1 + dist * prm[13]
            pre2 = av2 + bv2 + c2 + dist * prm[14]
            m0 = pre0 / (1.0 + jnp.exp(-pre0))
            m1 = pre1 / (1.0 + jnp.exp(-pre1))
            m2 = pre2 / (1.0 + jnp.exp(-pre2))

            q0 = m0 * prm[0] + m1 * prm[3] + m2 * prm[6] + prm[9]
            q1 = m0 * prm[1] + m1 * prm[4] + m2 * prm[7] + prm[10]
            q2 = m0 * prm[2] + m1 * prm[5] + m2 * prm[8] + prm[11]
            n0 = q0 / (1.0 + jnp.exp(-q0))
            n1 = q1 / (1.0 + jnp.exp(-q1))
            n2 = q2 / (1.0 + jnp.exp(-q2))

            mask = (li + (e_base + j * CHUNK)) < N_EDGES
            plsc.addupdate_scatter(a0, [ridx], n0, mask=mask)
            plsc.addupdate_scatter(a1, [ridx], n1, mask=mask)
            plsc.addupdate_scatter(a2, [ridx], n2, mask=mask)
            plsc.addupdate_scatter(a3, [ridx], n0 * rx, mask=mask)
            plsc.addupdate_scatter(a4, [ridx], n1 * ry, mask=mask)
            plsc.addupdate_scatter(a5, [ridx], n2 * rz, mask=mask)

    @pl.loop(0, NCHUNK // 2)
    def _pair(i):
        j0 = i * 2
        j1 = j0 + 1
        pltpu.async_copy(table_hbm.at[rowi.at[j1]], rb1, sem_r1)
        pltpu.async_copy(table_hbm.at[coli.at[j1]], cb1, sem_c1)
        pltpu.make_async_copy(table_hbm.at[rowi.at[j0]], rb0, sem_r0).wait()
        pltpu.make_async_copy(table_hbm.at[coli.at[j0]], cb0, sem_c0).wait()
        _compute(j0, rb0, cb0)
        jn = jnp.minimum(j0 + 2, NCHUNK - 1)
        pltpu.async_copy(table_hbm.at[rowi.at[jn]], rb0, sem_r0)
        pltpu.async_copy(table_hbm.at[coli.at[jn]], cb0, sem_c0)
        pltpu.make_async_copy(table_hbm.at[rowi.at[j1]], rb1, sem_r1).wait()
        pltpu.make_async_copy(table_hbm.at[coli.at[j1]], cb1, sem_c1).wait()
        _compute(j1, rb1, cb1)

    pltpu.make_async_copy(table_hbm.at[rowi.at[0]], rb0, sem_r0).wait()
    pltpu.make_async_copy(table_hbm.at[coli.at[0]], cb0, sem_c0).wait()

    pltpu.sync_copy(a0, out_hbm.at[wid, 0])
    pltpu.sync_copy(a1, out_hbm.at[wid, 1])
    pltpu.sync_copy(a2, out_hbm.at[wid, 2])
    pltpu.sync_copy(a3, out_hbm.at[wid, 3])
    pltpu.sync_copy(a4, out_hbm.at[wid, 4])
    pltpu.sync_copy(a5, out_hbm.at[wid, 5])


def _sc_edges(rowp, colp, table, ct, prm):
    mesh = plsc.VectorSubcoreMesh(core_axis_name="c", subcore_axis_name="s",
                                  num_cores=2, num_subcores=16)
    f = pl.kernel(
        _sc_body,
        out_type=jax.ShapeDtypeStruct((NW, 6, N_NODES), jnp.float32),
        mesh=mesh,
        compiler_params=pltpu.CompilerParams(needs_layout_passes=False,
                                             use_tc_tiling_on_sc=False),
        scratch_types=[
            pltpu.VMEM((NCHUNK, CHUNK), jnp.int32),
            pltpu.VMEM((NCHUNK, CHUNK), jnp.int32),
            pltpu.VMEM((24, CW), jnp.float32),
            pltpu.VMEM((16, 16), jnp.float32),
            pltpu.VMEM((CHUNK, 16), jnp.float32),
            pltpu.VMEM((CHUNK, 16), jnp.float32),
            pltpu.VMEM((CHUNK, 16), jnp.float32),
            pltpu.VMEM((CHUNK, 16), jnp.float32),
            pltpu.VMEM((N_NODES,), jnp.float32),
            pltpu.VMEM((N_NODES,), jnp.float32),
            pltpu.VMEM((N_NODES,), jnp.float32),
            pltpu.VMEM((N_NODES,), jnp.float32),
            pltpu.VMEM((N_NODES,), jnp.float32),
            pltpu.VMEM((N_NODES,), jnp.float32),
            pltpu.SemaphoreType.DMA,
            pltpu.SemaphoreType.DMA,
            pltpu.SemaphoreType.DMA,
            pltpu.SemaphoreType.DMA,
        ],
    )
    return f(rowp, colp, table, ct, prm)


def _node_body(x_ref, parts_ref, w1x_ref, w1m_ref, b1_ref,
               w2_ref, b2_ref, out_ref):
    acc = jnp.sum(parts_ref[...], axis=0)
    aggm_t = acc[0:3]
    aggp_t = acc[3:6]
    x = x_ref[...]
    z = (jnp.dot(x, w1x_ref[...], preferred_element_type=jnp.float32)
         + lax.dot_general(aggm_t, w1m_ref[...], (((0,), (0,)), ((), ())),
                           preferred_element_type=jnp.float32)
         + b1_ref[...])
    h2 = z * jax.nn.sigmoid(z)
    d2 = jnp.dot(h2, w2_ref[...],
                 preferred_element_type=jnp.float32) + b2_ref[...]
    dpos = lax.dot_general(aggp_t * 0.1, jnp.eye(3, dtype=jnp.float32),
                           (((0,), (0,)), ((), ())),
                           preferred_element_type=jnp.float32)
    out_ref[...] = x + jnp.concatenate([dpos, d2], axis=1)


def _node_mlp(x, parts, w1x, w1m, b1, w2, b2):
    blk = 2048
    grid = (N_NODES + blk - 1) // blk
    d_all = POS_DIM + FEATS
    return pl.pallas_call(
        _node_body,
        grid=(grid,),
        in_specs=[
            pl.BlockSpec((blk, d_all), lambda i: (i, 0)),
            pl.BlockSpec((NW, 6, blk), lambda i: (0, 0, i)),
            pl.BlockSpec((d_all, FEATS), lambda i: (0, 0)),
            pl.BlockSpec((3, FEATS), lambda i: (0, 0)),
            pl.BlockSpec((1, FEATS), lambda i: (0, 0)),
            pl.BlockSpec((FEATS, FEATS), lambda i: (0, 0)),
            pl.BlockSpec((1, FEATS), lambda i: (0, 0)),
        ],
        out_specs=pl.BlockSpec((blk, d_all), lambda i: (i, 0)),
        out_shape=jax.ShapeDtypeStruct((N_NODES, d_all), jnp.float32),
    )(x, parts, w1x, w1m, b1, w2, b2)


@jax.jit
def kernel(x, edge_index, edge_attr, batch, eW1, eb1, eW2, eb2,
           nW1, nb1, nW2, nb2):
    row = edge_index[0].astype(jnp.int32)
    col = edge_index[1].astype(jnp.int32)

    w6 = jnp.concatenate([eW1[:FEATS], eW1[FEATS:2 * FEATS]], axis=1)
    w6p = jnp.concatenate([jnp.zeros((POS_DIM, 6), jnp.float32), w6], axis=0)
    table = _node_proj(x, w6p)

    attr2d = edge_attr.reshape(A_ROWS, 128)
    wa = eW1[2 * FEATS:2 * FEATS + 16]
    wp = jnp.kron(jnp.eye(8, dtype=jnp.float32), wa)
    b24 = jnp.tile(eb1, 8).reshape(24, 1)
    ct = _edge_proj_t(attr2d, wp, b24)

    pad = E_PAD - N_EDGES
    rowp = jnp.pad(row, (0, pad)).reshape(NW, NCHUNK, CHUNK)
    colp = jnp.pad(col, (0, pad)).reshape(NW, NCHUNK, CHUNK)

    p = jnp.concatenate([eW2.reshape(-1), eb2, eW1[2 * FEATS + 16]])
    prm = jnp.broadcast_to(jnp.pad(p, (0, 1))[:, None], (16, 16))

    parts = _sc_edges(rowp, colp, table, ct, prm)

    w1x = jnp.concatenate([jnp.zeros((POS_DIM, FEATS), jnp.float32),
                           nW1[:FEATS]], axis=0)
    return _node_mlp(x, parts, w1x, nW1[FEATS:], nb1.reshape(1, -1),
                     nW2, nb2.reshape(1, -1))


# speedup vs baseline: 1.6439x; 1.0011x over previous
import jax
import jax.numpy as jnp
from jax import lax
from jax.experimental import pallas as pl
from jax.experimental.pallas import tpu as pltpu
from jax.experimental.pallas import tpu_sc as plsc

N_NODES = 10000
N_EDGES = 160000
POS_DIM = 3
FEATS = 256

NW = 32
CHUNK = 128
NCHUNK = 40
E_W = CHUNK * NCHUNK
E_PAD = NW * E_W


def _ab_body(x_ref, w6_ref, out_ref):
    x = x_ref[...]
    ab = jnp.dot(x, w6_ref[...], preferred_element_type=jnp.float32)
    pad = jnp.zeros((x.shape[0], 7), jnp.float32)
    out_ref[...] = jnp.concatenate([x[:, 0:POS_DIM], ab, pad], axis=1)


def _node_proj(x, w6p):
    blk = 2000
    grid = (N_NODES + blk - 1) // blk
    return pl.pallas_call(
        _ab_body,
        grid=(grid,),
        in_specs=[pl.BlockSpec((blk, POS_DIM + FEATS), lambda i: (i, 0)),
                  pl.BlockSpec((POS_DIM + FEATS, 6), lambda i: (0, 0))],
        out_specs=pl.BlockSpec((blk, 16), lambda i: (i, 0)),
        out_shape=jax.ShapeDtypeStruct((N_NODES, 16), jnp.float32),
    )(x, w6p)


A_ROWS = N_EDGES * 16 // 128
A_COLS = E_PAD * 16 // 128


def _ct_body(wp_ref, attr_ref, b_ref, out_ref):
    out_ref[...] = lax.dot_general(
        wp_ref[...], attr_ref[...], (((0,), (1,)), ((), ())),
        preferred_element_type=jnp.float32) + b_ref[...]


def _edge_proj_t(attr2d, wp, b24):
    blk = 2048
    grid = (A_ROWS + blk - 1) // blk
    return pl.pallas_call(
        _ct_body,
        grid=(grid,),
        in_specs=[pl.BlockSpec((128, 24), lambda i: (0, 0)),
                  pl.BlockSpec((blk, 128), lambda i: (i, 0)),
                  pl.BlockSpec((24, 1), lambda i: (0, 0))],
        out_specs=pl.BlockSpec((24, blk), lambda i: (0, i)),
        out_shape=jax.ShapeDtypeStruct((24, A_COLS), jnp.float32),
    )(wp, attr2d, b24)


CW = E_W // 8


def _sc_body(row_hbm, col_hbm, table_hbm, ct_hbm, prm_hbm, out_hbm,
             rowi, coli, cst, prm, rb0, cb0, rb1, cb1,
             a0, a1, a2, a3, a4, a5, sem_r0, sem_c0, sem_r1, sem_c1):
    nc = 2
    wid = lax.axis_index("s") * nc + lax.axis_index("c")

    pltpu.sync_copy(row_hbm.at[wid], rowi)
    pltpu.sync_copy(col_hbm.at[wid], coli)
    pltpu.async_copy(table_hbm.at[rowi.at[0]], rb0, sem_r0)
    pltpu.async_copy(table_hbm.at[coli.at[0]], cb0, sem_c0)
    pltpu.sync_copy(ct_hbm.at[:, pl.ds(wid * CW, CW)], cst)
    pltpu.sync_copy(prm_hbm, prm)

    zeros16 = jnp.zeros((16,), jnp.float32)

    @pl.loop(0, N_NODES // 16)
    def _zero(i):
        sl = pl.ds(i * 16, 16)
        a0[sl] = zeros16
        a1[sl] = zeros16
        a2[sl] = zeros16
        a3[sl] = zeros16
        a4[sl] = zeros16
        a5[sl] = zeros16

    li0 = lax.iota(jnp.int32, 16)
    cols = [jnp.full((16,), k, jnp.int32) for k in range(16)]
    e_base = wid * E_W
    row3 = (li0 & 7) * 3
    cshift = li0 >> 3

    def _compute(j, rbuf, cbuf):
        for i in range(8):
            li = li0 + (i * 16)
            ridx = rowi[j, pl.ds(i * 16, 16)]
            pxr = plsc.load_gather(rbuf, [li, cols[0]])
            pyr = plsc.load_gather(rbuf, [li, cols[1]])
            pzr = plsc.load_gather(rbuf, [li, cols[2]])
            av0 = plsc.load_gather(rbuf, [li, cols[3]])
            av1 = plsc.load_gather(rbuf, [li, cols[4]])
            av2 = plsc.load_gather(rbuf, [li, cols[5]])
            pxc = plsc.load_gather(cbuf, [li, cols[0]])
            pyc = plsc.load_gather(cbuf, [li, cols[1]])
            pzc = plsc.load_gather(cbuf, [li, cols[2]])
            bv0 = plsc.load_gather(cbuf, [li, cols[6]])
            bv1 = plsc.load_gather(cbuf, [li, cols[7]])
            bv2 = plsc.load_gather(cbuf, [li, cols[8]])
            colv = cshift + (j * (CHUNK // 8) + i * 2)
            c0 = plsc.load_gather(cst, [row3, colv])
            c1 = plsc.load_gather(cst, [row3 + 1, colv])
            c2 = plsc.load_gather(cst, [row3 + 2, colv])

            rx = pxr - pxc
            ry = pyr - pyc
            rz = pzr - pzc
            dist = rx * rx + ry * ry + rz * rz

            pre0 = av0 + bv0 + c0 + dist * prm[12]
            pre1 = av1 + bv1 + c1 + dist * prm[13]
            pre2 = av2 + bv2 + c2 + dist * prm[14]
            m0 = pre0 / (1.0 + jnp.exp(-pre0))
            m1 = pre1 / (1.0 + jnp.exp(-pre1))
            m2 = pre2 / (1.0 + jnp.exp(-pre2))

            q0 = m0 * prm[0] + m1 * prm[3] + m2 * prm[6] + prm[9]
            q1 = m0 * prm[1] + m1 * prm[4] + m2 * prm[7] + prm[10]
            q2 = m0 * prm[2] + m1 * prm[5] + m2 * prm[8] + prm[11]
            n0 = q0 / (1.0 + jnp.exp(-q0))
            n1 = q1 / (1.0 + jnp.exp(-q1))
            n2 = q2 / (1.0 + jnp.exp(-q2))

            mask = (li + (e_base + j * CHUNK)) < N_EDGES
            plsc.addupdate_scatter(a0, [ridx], n0, mask=mask)
            plsc.addupdate_scatter(a1, [ridx], n1, mask=mask)
            plsc.addupdate_scatter(a2, [ridx], n2, mask=mask)
            plsc.addupdate_scatter(a3, [ridx], n0 * rx, mask=mask)
            plsc.addupdate_scatter(a4, [ridx], n1 * ry, mask=mask)
            plsc.addupdate_scatter(a5, [ridx], n2 * rz, mask=mask)

    @pl.loop(0, NCHUNK // 2)
    def _pair(i):
        j0 = i * 2
        j1 = j0 + 1
        pltpu.async_copy(table_hbm.at[rowi.at[j1]], rb1, sem_r1)
        pltpu.async_copy(table_hbm.at[coli.at[j1]], cb1, sem_c1)
        pltpu.make_async_copy(table_hbm.at[rowi.at[j0]], rb0, sem_r0).wait()
        pltpu.make_async_copy(table_hbm.at[coli.at[j0]], cb0, sem_c0).wait()
        _compute(j0, rb0, cb0)
        jn = jnp.minimum(j0 + 2, NCHUNK - 1)
        pltpu.async_copy(table_hbm.at[rowi.at[jn]], rb0, sem_r0)
        pltpu.async_copy(table_hbm.at[coli.at[jn]], cb0, sem_c0)
        pltpu.make_async_copy(table_hbm.at[rowi.at[j1]], rb1, sem_r1).wait()
        pltpu.make_async_copy(table_hbm.at[coli.at[j1]], cb1, sem_c1).wait()
        _compute(j1, rb1, cb1)

    pltpu.make_async_copy(table_hbm.at[rowi.at[0]], rb0, sem_r0).wait()
    pltpu.make_async_copy(table_hbm.at[coli.at[0]], cb0, sem_c0).wait()

    pltpu.sync_copy(a0, out_hbm.at[wid, 0])
    pltpu.sync_copy(a1, out_hbm.at[wid, 1])
    pltpu.sync_copy(a2, out_hbm.at[wid, 2])
    pltpu.sync_copy(a3, out_hbm.at[wid, 3])
    pltpu.sync_copy(a4, out_hbm.at[wid, 4])
    pltpu.sync_copy(a5, out_hbm.at[wid, 5])


def _sc_edges(rowp, colp, table, ct, prm):
    mesh = plsc.VectorSubcoreMesh(core_axis_name="c", subcore_axis_name="s",
                                  num_cores=2, num_subcores=16)
    f = pl.kernel(
        _sc_body,
        out_type=jax.ShapeDtypeStruct((NW, 6, N_NODES), jnp.float32),
        mesh=mesh,
        compiler_params=pltpu.CompilerParams(needs_layout_passes=False,
                                             use_tc_tiling_on_sc=False),
        scratch_types=[
            pltpu.VMEM((NCHUNK, CHUNK), jnp.int32),
            pltpu.VMEM((NCHUNK, CHUNK), jnp.int32),
            pltpu.VMEM((24, CW), jnp.float32),
            pltpu.VMEM((16, 16), jnp.float32),
            pltpu.VMEM((CHUNK, 16), jnp.float32),
            pltpu.VMEM((CHUNK, 16), jnp.float32),
            pltpu.VMEM((CHUNK, 16), jnp.float32),
            pltpu.VMEM((CHUNK, 16), jnp.float32),
            pltpu.VMEM((N_NODES,), jnp.float32),
            pltpu.VMEM((N_NODES,), jnp.float32),
            pltpu.VMEM((N_NODES,), jnp.float32),
            pltpu.VMEM((N_NODES,), jnp.float32),
            pltpu.VMEM((N_NODES,), jnp.float32),
            pltpu.VMEM((N_NODES,), jnp.float32),
            pltpu.SemaphoreType.DMA,
            pltpu.SemaphoreType.DMA,
            pltpu.SemaphoreType.DMA,
            pltpu.SemaphoreType.DMA,
        ],
    )
    return f(rowp, colp, table, ct, prm)


def _node_body(x_ref, parts_ref, w1x_ref, w1m_ref, b1_ref,
               w2_ref, b2_ref, out_ref):
    acc = jnp.sum(parts_ref[...], axis=0)
    aggm_t = acc[0:3]
    aggp_t = acc[3:6]
    x = x_ref[...]
    z = (jnp.dot(x, w1x_ref[...], preferred_element_type=jnp.float32)
         + lax.dot_general(aggm_t, w1m_ref[...], (((0,), (0,)), ((), ())),
                           preferred_element_type=jnp.float32)
         + b1_ref[...])
    h2 = z * jax.nn.sigmoid(z)
    d2 = jnp.dot(h2, w2_ref[...],
                 preferred_element_type=jnp.float32) + b2_ref[...]
    dpos = lax.dot_general(aggp_t * 0.1, jnp.eye(3, dtype=jnp.float32),
                           (((0,), (0,)), ((), ())),
                           preferred_element_type=jnp.float32)
    out_ref[...] = x + jnp.concatenate([dpos, d2], axis=1)


def _node_mlp(x, parts, w1x, w1m, b1, w2, b2):
    blk = 2048
    grid = (N_NODES + blk - 1) // blk
    d_all = POS_DIM + FEATS
    return pl.pallas_call(
        _node_body,
        grid=(grid,),
        in_specs=[
            pl.BlockSpec((blk, d_all), lambda i: (i, 0)),
            pl.BlockSpec((NW, 6, blk), lambda i: (0, 0, i)),
            pl.BlockSpec((d_all, FEATS), lambda i: (0, 0)),
            pl.BlockSpec((3, FEATS), lambda i: (0, 0)),
            pl.BlockSpec((1, FEATS), lambda i: (0, 0)),
            pl.BlockSpec((FEATS, FEATS), lambda i: (0, 0)),
            pl.BlockSpec((1, FEATS), lambda i: (0, 0)),
        ],
        out_specs=pl.BlockSpec((blk, d_all), lambda i: (i, 0)),
        out_shape=jax.ShapeDtypeStruct((N_NODES, d_all), jnp.float32),
    )(x, parts, w1x, w1m, b1, w2, b2)


@jax.jit
def kernel(x, edge_index, edge_attr, batch, eW1, eb1, eW2, eb2,
           nW1, nb1, nW2, nb2):
    row = edge_index[0].astype(jnp.int32)
    col = edge_index[1].astype(jnp.int32)

    w6 = jnp.concatenate([eW1[:FEATS], eW1[FEATS:2 * FEATS]], axis=1)
    w6p = jnp.concatenate([jnp.zeros((POS_DIM, 6), jnp.float32), w6], axis=0)
    table = _node_proj(x, w6p)

    attr2d = edge_attr.reshape(A_ROWS, 128)
    wa = eW1[2 * FEATS:2 * FEATS + 16]
    wp = jnp.kron(jnp.eye(8, dtype=jnp.float32), wa)
    b24 = jnp.tile(eb1, 8).reshape(24, 1)
    ct = _edge_proj_t(attr2d, wp, b24)

    pad = E_PAD - N_EDGES
    rowp = jnp.pad(row, (0, pad)).reshape(NW, NCHUNK, CHUNK)
    colp = jnp.pad(col, (0, pad)).reshape(NW, NCHUNK, CHUNK)

    p = jnp.concatenate([eW2.reshape(-1), eb2, eW1[2 * FEATS + 16]])
    prm = jnp.broadcast_to(jnp.pad(p, (0, 1))[:, None], (16, 16))

    parts = _sc_edges(rowp, colp, table, ct, prm)

    w1x = jnp.concatenate([jnp.zeros((POS_DIM, FEATS), jnp.float32),
                           nW1[:FEATS]], axis=0)
    return _node_mlp(x, parts, w1x, nW1[FEATS:], nb1.reshape(1, -1),
                     nW2, nb2.reshape(1, -1))
